# baseline (speedup 1.0000x reference)
import functools

import jax
import jax.numpy as jnp
from jax import lax
from jax.experimental import pallas as pl
from jax.experimental.pallas import tpu as pltpu
from jax.experimental.pallas import tpu_sc as plsc

NC, NS, L = 2, 16, 16
NW = NC * NS
E = 320000
EPW = E // NW
CH = 128
NFULL = EPW // CH
TAIL = EPW - NFULL * CH
NCH = NFULL + 1
IPAD = NCH * CH
D = 128
DW = D // 2
H = 10000


PP = 125


def _dot_body(h1, eidx, out, hs, idx_s, idx_d, rows_s, rows_d, out_v,
              pin, pout, sems):
  sid = lax.axis_index("s")
  wid = sid * NC + lax.axis_index("c")
  base = wid * EPW

  rp = H // NS
  for q in range(rp // PP):
    r0 = sid * rp + q * PP
    pltpu.sync_copy(h1.at[pl.ds(r0 * D, PP * D)], pin)

    @plsc.parallel_loop(0, PP, step=1, unroll=2)
    def prow(r):
      for j in range(DW // L):
        a = pin[pl.ds(r * D + j * L, L)]
        bb = pin[pl.ds(r * D + DW + j * L, L)]
        w = plsc.bitcast(
            plsc.pack(a, bb, format=plsc.PackFormat.INTERLEAVED), jnp.int32)
        pout[r, pl.ds(j * L, L)] = w

    pltpu.sync_copy(pout, hs.at[pl.ds(r0, PP)])
  plsc.subcore_barrier()

  zeros = jnp.zeros((L,), jnp.int32)
  for i in range((IPAD - EPW) // L):
    idx_s[pl.ds(EPW + i * L, L)] = zeros
    idx_d[pl.ds(EPW + i * L, L)] = zeros
  pltpu.sync_copy(eidx.at[0, pl.ds(base, EPW)], idx_s.at[pl.ds(0, EPW)])
  pltpu.sync_copy(eidx.at[1, pl.ds(base, EPW)], idx_d.at[pl.ds(0, EPW)])

  def fire(g, b):
    isl = idx_s.at[pl.ds(g * CH, CH)]
    idl = idx_d.at[pl.ds(g * CH, CH)]
    pltpu.async_copy(hs.at[isl], rows_s.at[b], sems.at[2 * b])
    pltpu.async_copy(hs.at[idl], rows_d.at[b], sems.at[2 * b + 1])

  def drain(g, b):
    isl = idx_s.at[pl.ds(g * CH, CH)]
    idl = idx_d.at[pl.ds(g * CH, CH)]
    pltpu.make_async_copy(hs.at[isl], rows_s.at[b], sems.at[2 * b]).wait()
    pltpu.make_async_copy(hs.at[idl], rows_d.at[b], sems.at[2 * b + 1]).wait()

  lane = lax.iota(jnp.int32, L)
  last = lane == (L - 1)

  def compute(g, b, n):
    rs = rows_s.at[b]
    rd = rows_d.at[b]

    @plsc.parallel_loop(0, n, step=1, unroll=8)
    def edge(e):
      p = None
      for j in range(DW // L):
        a = plsc.bitcast(rs[e, pl.ds(j * L, L)], jnp.bfloat16)
        bb = plsc.bitcast(rd[e, pl.ds(j * L, L)], jnp.bfloat16)
        t = a * bb
        p = t if p is None else p + t
      lo, hi = plsc.unpack(p, format=plsc.PackFormat.INTERLEAVED)
      tot = plsc.cumsum(lo + hi)
      plsc.store_compressed(out_v.at[pl.ds(g * CH + e, L)], tot, mask=last)

  fire(0, 0)

  def step(gg, carry):
    for b in range(2):
      g = 2 * gg + b
      fire(g + 1, 1 - b)
      drain(g, b)
      compute(g, b, CH)
    return carry

  lax.fori_loop(0, NFULL // 2, step, 0)
  drain(NFULL, 0)
  compute(NFULL, 0, TAIL)

  pltpu.sync_copy(out_v.at[pl.ds(0, EPW)], out.at[pl.ds(base, EPW)])


_dot_sc = functools.partial(
    pl.kernel,
    out_type=jax.ShapeDtypeStruct((E,), jnp.float32),
    mesh=plsc.VectorSubcoreMesh(
        core_axis_name="c", subcore_axis_name="s",
        num_cores=NC, num_subcores=NS),
    compiler_params=pltpu.CompilerParams(
        needs_layout_passes=False, use_tc_tiling_on_sc=False),
    scratch_types=[
        pltpu.VMEM_SHARED((H, DW), jnp.int32),
        pltpu.VMEM((IPAD,), jnp.int32),
        pltpu.VMEM((IPAD,), jnp.int32),
        pltpu.VMEM((2, CH, DW), jnp.int32),
        pltpu.VMEM((2, CH, DW), jnp.int32),
        pltpu.VMEM((EPW + L,), jnp.float32),
        pltpu.VMEM((PP * D,), jnp.float32),
        pltpu.VMEM((PP, DW), jnp.int32),
        pltpu.SemaphoreType.DMA((4,)),
    ],
)(_dot_body)


@jax.jit
def kernel(h, edge_index):
  return _dot_sc(h.reshape(-1), edge_index)


# speedup vs baseline: 1.0432x; 1.0432x over previous
import functools

import jax
import jax.numpy as jnp
from jax import lax
from jax.experimental import pallas as pl
from jax.experimental.pallas import tpu as pltpu
from jax.experimental.pallas import tpu_sc as plsc

NC, NS, L = 2, 16, 16
NW = NC * NS
E = 320000
EPW = E // NW
CH = 128
NFULL = EPW // CH
TAIL = EPW - NFULL * CH
NCH = NFULL + 1
IPAD = NCH * CH
D = 128
DW = D // 2
H = 10000


def _dot_body(hb, eidx, out, hs, idx_s, idx_d, rows_s, rows_d, out_v, sems):
  sid = lax.axis_index("s")
  wid = sid * NC + lax.axis_index("c")
  base = wid * EPW

  rp = H // NS
  pltpu.async_copy(hb.at[pl.ds(sid * rp, rp)], hs.at[pl.ds(sid * rp, rp)],
                   sems.at[4])
  pltpu.async_copy(eidx.at[0, pl.ds(base, EPW)], idx_s.at[pl.ds(0, EPW)],
                   sems.at[5])
  pltpu.async_copy(eidx.at[1, pl.ds(base, EPW)], idx_d.at[pl.ds(0, EPW)],
                   sems.at[5])
  zeros = jnp.zeros((L,), jnp.int32)
  for i in range((IPAD - EPW) // L):
    idx_s[pl.ds(EPW + i * L, L)] = zeros
    idx_d[pl.ds(EPW + i * L, L)] = zeros
  pltpu.make_async_copy(
      eidx.at[0, pl.ds(base, EPW)], idx_s.at[pl.ds(0, EPW)], sems.at[5]).wait()
  pltpu.make_async_copy(
      eidx.at[1, pl.ds(base, EPW)], idx_d.at[pl.ds(0, EPW)], sems.at[5]).wait()
  pltpu.make_async_copy(
      hb.at[pl.ds(sid * rp, rp)], hs.at[pl.ds(sid * rp, rp)], sems.at[4]).wait()
  plsc.subcore_barrier()

  def fire(g, b):
    isl = idx_s.at[pl.ds(g * CH, CH)]
    idl = idx_d.at[pl.ds(g * CH, CH)]
    pltpu.async_copy(hs.at[isl], rows_s.at[b], sems.at[2 * b])
    pltpu.async_copy(hs.at[idl], rows_d.at[b], sems.at[2 * b + 1])

  def drain(g, b):
    isl = idx_s.at[pl.ds(g * CH, CH)]
    idl = idx_d.at[pl.ds(g * CH, CH)]
    pltpu.make_async_copy(hs.at[isl], rows_s.at[b], sems.at[2 * b]).wait()
    pltpu.make_async_copy(hs.at[idl], rows_d.at[b], sems.at[2 * b + 1]).wait()

  lane = lax.iota(jnp.int32, L)
  last = lane == (L - 1)

  def compute(g, b, n):
    rs = rows_s.at[b]
    rd = rows_d.at[b]

    @plsc.parallel_loop(0, n, step=1, unroll=8)
    def edge(e):
      p = None
      for j in range(DW // L):
        a = plsc.bitcast(rs[e, pl.ds(j * L, L)], jnp.bfloat16)
        bb = plsc.bitcast(rd[e, pl.ds(j * L, L)], jnp.bfloat16)
        t = a * bb
        p = t if p is None else p + t
      lo, hi = plsc.unpack(p, format=plsc.PackFormat.INTERLEAVED)
      tot = plsc.cumsum(lo + hi)
      plsc.store_compressed(out_v.at[pl.ds(g * CH + e, L)], tot, mask=last)

  fire(0, 0)
  fire(1, 1)

  def step(gg, carry):
    for b in range(3):
      g = 3 * gg + b

      @pl.when(g + 2 < NCH)
      def _():
        fire(g + 2, (b + 2) % 3)

      drain(g, b)
      compute(g, b, CH)
    return carry

  lax.fori_loop(0, NFULL // 3, step, 0)
  drain(NFULL, NFULL % 3)
  compute(NFULL, NFULL % 3, TAIL)

  pltpu.sync_copy(out_v.at[pl.ds(0, EPW)], out.at[pl.ds(base, EPW)])


_dot_sc = functools.partial(
    pl.kernel,
    out_type=jax.ShapeDtypeStruct((E,), jnp.float32),
    mesh=plsc.VectorSubcoreMesh(
        core_axis_name="c", subcore_axis_name="s",
        num_cores=NC, num_subcores=NS),
    compiler_params=pltpu.CompilerParams(
        needs_layout_passes=False, use_tc_tiling_on_sc=False),
    scratch_types=[
        pltpu.VMEM_SHARED((H, DW), jnp.int32),
        pltpu.VMEM((IPAD,), jnp.int32),
        pltpu.VMEM((IPAD,), jnp.int32),
        pltpu.VMEM((3, CH, DW), jnp.int32),
        pltpu.VMEM((3, CH, DW), jnp.int32),
        pltpu.VMEM((EPW + L,), jnp.float32),
        pltpu.SemaphoreType.DMA((6,)),
    ],
)(_dot_body)


@jax.jit
def kernel(h, edge_index):
  lo = lax.bitcast_convert_type(h[:, :DW].astype(jnp.bfloat16), jnp.uint16)
  hi = lax.bitcast_convert_type(h[:, DW:].astype(jnp.bfloat16), jnp.uint16)
  hb = lax.bitcast_convert_type(
      lo.astype(jnp.uint32) | (hi.astype(jnp.uint32) << 16), jnp.int32)
  return _dot_sc(hb, edge_index)


# speedup vs baseline: 1.0801x; 1.0354x over previous
import functools

import jax
import jax.numpy as jnp
from jax import lax
from jax.experimental import pallas as pl
from jax.experimental.pallas import tpu as pltpu
from jax.experimental.pallas import tpu_sc as plsc

NC, NS, L = 2, 16, 16
NW = NC * NS
E = 320000
EPW = E // NW
CH = 128
NFULL = EPW // CH
TAIL = EPW - NFULL * CH
NCH = NFULL + 1
IPAD = NCH * CH
D = 128
DW = D // 2
H = 10000


def _dot_body(hb, eidx, out, hs, idx_s, idx_d, rows_s, rows_d, out_v, sems):
  sid = lax.axis_index("s")
  wid = sid * NC + lax.axis_index("c")
  base = wid * EPW

  rp = H // NS
  pltpu.async_copy(hb.at[pl.ds(sid * rp, rp)], hs.at[pl.ds(sid * rp, rp)],
                   sems.at[4])
  pltpu.async_copy(eidx.at[0, pl.ds(base, EPW)], idx_s.at[pl.ds(0, EPW)],
                   sems.at[5])
  pltpu.async_copy(eidx.at[1, pl.ds(base, EPW)], idx_d.at[pl.ds(0, EPW)],
                   sems.at[5])
  zeros = jnp.zeros((L,), jnp.int32)
  for i in range((IPAD - EPW) // L):
    idx_s[pl.ds(EPW + i * L, L)] = zeros
    idx_d[pl.ds(EPW + i * L, L)] = zeros
  pltpu.make_async_copy(
      eidx.at[0, pl.ds(base, EPW)], idx_s.at[pl.ds(0, EPW)], sems.at[5]).wait()
  pltpu.make_async_copy(
      eidx.at[1, pl.ds(base, EPW)], idx_d.at[pl.ds(0, EPW)], sems.at[5]).wait()
  pltpu.make_async_copy(
      hb.at[pl.ds(sid * rp, rp)], hs.at[pl.ds(sid * rp, rp)], sems.at[4]).wait()
  plsc.subcore_barrier()

  def fire(g, b):
    isl = idx_s.at[pl.ds(g * CH, CH)]
    idl = idx_d.at[pl.ds(g * CH, CH)]
    pltpu.async_copy(hs.at[isl], rows_s.at[b], sems.at[2 * b])
    pltpu.async_copy(hs.at[idl], rows_d.at[b], sems.at[2 * b + 1])

  def drain(g, b):
    isl = idx_s.at[pl.ds(g * CH, CH)]
    idl = idx_d.at[pl.ds(g * CH, CH)]
    pltpu.make_async_copy(hs.at[isl], rows_s.at[b], sems.at[2 * b]).wait()
    pltpu.make_async_copy(hs.at[idl], rows_d.at[b], sems.at[2 * b + 1]).wait()

  lane = lax.iota(jnp.int32, L)
  last = lane == (L - 1)

  def compute(g, b, n):
    rs = rows_s.at[b]
    rd = rows_d.at[b]

    @plsc.parallel_loop(0, n, step=1, unroll=4)
    def edge(e):
      p = None
      for j in range(DW // L):
        a = plsc.bitcast(rs[e, pl.ds(j * L, L)], jnp.bfloat16)
        bb = plsc.bitcast(rd[e, pl.ds(j * L, L)], jnp.bfloat16)
        t = a * bb
        p = t if p is None else p + t
      lo, hi = plsc.unpack(p, format=plsc.PackFormat.INTERLEAVED)
      tot = plsc.cumsum(lo + hi)
      plsc.store_compressed(out_v.at[pl.ds(g * CH + e, L)], tot, mask=last)

  fire(0, 0)
  fire(1, 1)

  def step(gg, carry):
    for b in range(3):
      g = 3 * gg + b

      @pl.when(g + 2 < NCH)
      def _():
        fire(g + 2, (b + 2) % 3)

      drain(g, b)
      compute(g, b, CH)
    return carry

  lax.fori_loop(0, NFULL // 3, step, 0)
  drain(NFULL, NFULL % 3)
  compute(NFULL, NFULL % 3, TAIL)

  pltpu.sync_copy(out_v.at[pl.ds(0, EPW)], out.at[pl.ds(base, EPW)])


_dot_sc = functools.partial(
    pl.kernel,
    out_type=jax.ShapeDtypeStruct((E,), jnp.float32),
    mesh=plsc.VectorSubcoreMesh(
        core_axis_name="c", subcore_axis_name="s",
        num_cores=NC, num_subcores=NS),
    compiler_params=pltpu.CompilerParams(
        needs_layout_passes=False, use_tc_tiling_on_sc=False),
    scratch_types=[
        pltpu.VMEM_SHARED((H, DW), jnp.int32),
        pltpu.VMEM((IPAD,), jnp.int32),
        pltpu.VMEM((IPAD,), jnp.int32),
        pltpu.VMEM((3, CH, DW), jnp.int32),
        pltpu.VMEM((3, CH, DW), jnp.int32),
        pltpu.VMEM((EPW + L,), jnp.float32),
        pltpu.SemaphoreType.DMA((6,)),
    ],
)(_dot_body)


@jax.jit
def kernel(h, edge_index):
  lo = lax.bitcast_convert_type(h[:, :DW].astype(jnp.bfloat16), jnp.uint16)
  hi = lax.bitcast_convert_type(h[:, DW:].astype(jnp.bfloat16), jnp.uint16)
  hb = lax.bitcast_convert_type(
      lo.astype(jnp.uint32) | (hi.astype(jnp.uint32) << 16), jnp.int32)
  return _dot_sc(hb, edge_index)


# speedup vs baseline: 1.0901x; 1.0093x over previous
import functools

import jax
import jax.numpy as jnp
from jax import lax
from jax.experimental import pallas as pl
from jax.experimental.pallas import tpu as pltpu
from jax.experimental.pallas import tpu_sc as plsc

NC, NS, L = 2, 16, 16
NW = NC * NS
E = 320000
EPW = E // NW
CH = 128
NFULL = EPW // CH
TAIL = EPW - NFULL * CH
NCH = NFULL + 1
IPAD = NCH * CH
D = 128
DW = D // 2
H = 10000


def _dot_body(hb, eidx, out, hs, idx_s, idx_d, rows_s, rows_d, out_v, sems):
  sid = lax.axis_index("s")
  wid = sid * NC + lax.axis_index("c")
  base = wid * EPW

  rp = H // NS
  pltpu.async_copy(hb.at[pl.ds(sid * rp, rp)], hs.at[pl.ds(sid * rp, rp)],
                   sems.at[4])
  pltpu.async_copy(eidx.at[0, pl.ds(base, EPW)], idx_s.at[pl.ds(0, EPW)],
                   sems.at[5])
  pltpu.async_copy(eidx.at[1, pl.ds(base, EPW)], idx_d.at[pl.ds(0, EPW)],
                   sems.at[5])
  zeros = jnp.zeros((L,), jnp.int32)
  for i in range((IPAD - EPW) // L):
    idx_s[pl.ds(EPW + i * L, L)] = zeros
    idx_d[pl.ds(EPW + i * L, L)] = zeros
  pltpu.make_async_copy(
      eidx.at[0, pl.ds(base, EPW)], idx_s.at[pl.ds(0, EPW)], sems.at[5]).wait()
  pltpu.make_async_copy(
      eidx.at[1, pl.ds(base, EPW)], idx_d.at[pl.ds(0, EPW)], sems.at[5]).wait()
  pltpu.make_async_copy(
      hb.at[pl.ds(sid * rp, rp)], hs.at[pl.ds(sid * rp, rp)], sems.at[4]).wait()
  plsc.subcore_barrier()

  def fire(g, b):
    isl = idx_s.at[pl.ds(g * CH, CH)]
    idl = idx_d.at[pl.ds(g * CH, CH)]
    pltpu.async_copy(hs.at[isl], rows_s.at[b], sems.at[2 * b])
    pltpu.async_copy(hs.at[idl], rows_d.at[b], sems.at[2 * b + 1])

  def drain(g, b):
    isl = idx_s.at[pl.ds(g * CH, CH)]
    idl = idx_d.at[pl.ds(g * CH, CH)]
    pltpu.make_async_copy(hs.at[isl], rows_s.at[b], sems.at[2 * b]).wait()
    pltpu.make_async_copy(hs.at[idl], rows_d.at[b], sems.at[2 * b + 1]).wait()

  lane = lax.iota(jnp.int32, L)
  last = lane == (L - 1)

  def compute(g, b, n):
    rs = rows_s.at[b]
    rd = rows_d.at[b]

    @plsc.parallel_loop(0, n, step=1, unroll=2)
    def edge(e):
      p = None
      for j in range(DW // L):
        a = plsc.bitcast(rs[e, pl.ds(j * L, L)], jnp.bfloat16)
        bb = plsc.bitcast(rd[e, pl.ds(j * L, L)], jnp.bfloat16)
        t = a * bb
        p = t if p is None else p + t
      lo, hi = plsc.unpack(p, format=plsc.PackFormat.INTERLEAVED)
      tot = plsc.cumsum(lo + hi)
      plsc.store_compressed(out_v.at[pl.ds(g * CH + e, L)], tot, mask=last)

  fire(0, 0)
  fire(1, 1)

  def step(gg, carry):
    for b in range(3):
      g = 3 * gg + b

      @pl.when(g + 2 < NCH)
      def _():
        fire(g + 2, (b + 2) % 3)

      drain(g, b)
      compute(g, b, CH)
    return carry

  lax.fori_loop(0, NFULL // 3, step, 0)
  drain(NFULL, NFULL % 3)
  compute(NFULL, NFULL % 3, TAIL)

  pltpu.sync_copy(out_v.at[pl.ds(0, EPW)], out.at[pl.ds(base, EPW)])


_dot_sc = functools.partial(
    pl.kernel,
    out_type=jax.ShapeDtypeStruct((E,), jnp.float32),
    mesh=plsc.VectorSubcoreMesh(
        core_axis_name="c", subcore_axis_name="s",
        num_cores=NC, num_subcores=NS),
    compiler_params=pltpu.CompilerParams(
        needs_layout_passes=False, use_tc_tiling_on_sc=False),
    scratch_types=[
        pltpu.VMEM_SHARED((H, DW), jnp.int32),
        pltpu.VMEM((IPAD,), jnp.int32),
        pltpu.VMEM((IPAD,), jnp.int32),
        pltpu.VMEM((3, CH, DW), jnp.int32),
        pltpu.VMEM((3, CH, DW), jnp.int32),
        pltpu.VMEM((EPW + L,), jnp.float32),
        pltpu.SemaphoreType.DMA((6,)),
    ],
)(_dot_body)


@jax.jit
def kernel(h, edge_index):
  lo = lax.bitcast_convert_type(h[:, :DW].astype(jnp.bfloat16), jnp.uint16)
  hi = lax.bitcast_convert_type(h[:, DW:].astype(jnp.bfloat16), jnp.uint16)
  hb = lax.bitcast_convert_type(
      lo.astype(jnp.uint32) | (hi.astype(jnp.uint32) << 16), jnp.int32)
  return _dot_sc(hb, edge_index)


# speedup vs baseline: 1.1080x; 1.0164x over previous
import functools

import jax
import jax.numpy as jnp
from jax import lax
from jax.experimental import pallas as pl
from jax.experimental.pallas import tpu as pltpu
from jax.experimental.pallas import tpu_sc as plsc

NC, NS, L = 2, 16, 16
NW = NC * NS
E = 320000
EPW = E // NW
CH = 128
NFULL = EPW // CH
TAIL = EPW - NFULL * CH
NCH = NFULL + 1
IPAD = NCH * CH
D = 128
DW = D // 2
H = 10000


def _dot_body(hb, eidx, out, hs, idx_s, idx_d, rows_s, rows_d, out_v, sems):
  sid = lax.axis_index("s")
  wid = sid * NC + lax.axis_index("c")
  base = wid * EPW

  rp = H // NS
  pltpu.async_copy(hb.at[pl.ds(sid * rp, rp)], hs.at[pl.ds(sid * rp, rp)],
                   sems.at[4])
  pltpu.async_copy(eidx.at[0, pl.ds(base, EPW)], idx_s.at[pl.ds(0, EPW)],
                   sems.at[5])
  pltpu.async_copy(eidx.at[1, pl.ds(base, EPW)], idx_d.at[pl.ds(0, EPW)],
                   sems.at[5])
  zeros = jnp.zeros((L,), jnp.int32)
  for i in range((IPAD - EPW) // L):
    idx_s[pl.ds(EPW + i * L, L)] = zeros
    idx_d[pl.ds(EPW + i * L, L)] = zeros
  pltpu.make_async_copy(
      eidx.at[0, pl.ds(base, EPW)], idx_s.at[pl.ds(0, EPW)], sems.at[5]).wait()
  pltpu.make_async_copy(
      eidx.at[1, pl.ds(base, EPW)], idx_d.at[pl.ds(0, EPW)], sems.at[5]).wait()
  pltpu.make_async_copy(
      hb.at[pl.ds(sid * rp, rp)], hs.at[pl.ds(sid * rp, rp)], sems.at[4]).wait()
  plsc.subcore_barrier()

  def fire(g, b):
    isl = idx_s.at[pl.ds(g * CH, CH)]
    idl = idx_d.at[pl.ds(g * CH, CH)]
    pltpu.async_copy(hs.at[isl], rows_s.at[b], sems.at[2 * b])
    pltpu.async_copy(hs.at[idl], rows_d.at[b], sems.at[2 * b + 1])

  def drain(g, b):
    isl = idx_s.at[pl.ds(g * CH, CH)]
    idl = idx_d.at[pl.ds(g * CH, CH)]
    pltpu.make_async_copy(hs.at[isl], rows_s.at[b], sems.at[2 * b]).wait()
    pltpu.make_async_copy(hs.at[idl], rows_d.at[b], sems.at[2 * b + 1]).wait()

  lane = lax.iota(jnp.int32, L)
  last = lane == (L - 1)

  def compute(g, b, n):
    rs = rows_s.at[b]
    rd = rows_d.at[b]

    @plsc.parallel_loop(0, n, step=1, unroll=1)
    def edge(e):
      p = None
      for j in range(DW // L):
        a = plsc.bitcast(rs[e, pl.ds(j * L, L)], jnp.bfloat16)
        bb = plsc.bitcast(rd[e, pl.ds(j * L, L)], jnp.bfloat16)
        t = a * bb
        p = t if p is None else p + t
      lo, hi = plsc.unpack(p, format=plsc.PackFormat.INTERLEAVED)
      tot = plsc.cumsum(lo + hi)
      plsc.store_compressed(out_v.at[pl.ds(g * CH + e, L)], tot, mask=last)

  fire(0, 0)
  fire(1, 1)

  def step(gg, carry):
    for b in range(3):
      g = 3 * gg + b

      @pl.when(g + 2 < NCH)
      def _():
        fire(g + 2, (b + 2) % 3)

      drain(g, b)
      compute(g, b, CH)
    return carry

  lax.fori_loop(0, NFULL // 3, step, 0)
  drain(NFULL, NFULL % 3)
  compute(NFULL, NFULL % 3, TAIL)

  pltpu.sync_copy(out_v.at[pl.ds(0, EPW)], out.at[pl.ds(base, EPW)])


_dot_sc = functools.partial(
    pl.kernel,
    out_type=jax.ShapeDtypeStruct((E,), jnp.float32),
    mesh=plsc.VectorSubcoreMesh(
        core_axis_name="c", subcore_axis_name="s",
        num_cores=NC, num_subcores=NS),
    compiler_params=pltpu.CompilerParams(
        needs_layout_passes=False, use_tc_tiling_on_sc=False),
    scratch_types=[
        pltpu.VMEM_SHARED((H, DW), jnp.int32),
        pltpu.VMEM((IPAD,), jnp.int32),
        pltpu.VMEM((IPAD,), jnp.int32),
        pltpu.VMEM((3, CH, DW), jnp.int32),
        pltpu.VMEM((3, CH, DW), jnp.int32),
        pltpu.VMEM((EPW + L,), jnp.float32),
        pltpu.SemaphoreType.DMA((6,)),
    ],
)(_dot_body)


@jax.jit
def kernel(h, edge_index):
  lo = lax.bitcast_convert_type(h[:, :DW].astype(jnp.bfloat16), jnp.uint16)
  hi = lax.bitcast_convert_type(h[:, DW:].astype(jnp.bfloat16), jnp.uint16)
  hb = lax.bitcast_convert_type(
      lo.astype(jnp.uint32) | (hi.astype(jnp.uint32) << 16), jnp.int32)
  return _dot_sc(hb, edge_index)
